# baseline (speedup 1.0000x reference)
import dataclasses
import functools

import jax
import jax.numpy as jnp
from jax import lax
from jax.experimental import pallas as pl
from jax.experimental.pallas import tpu as pltpu
from jax.experimental.pallas import tpu_sc as plsc

N = 10000
E = 320000
D = 128
CHUNK = 128
NC, NS = 2, 16
NW = NC * NS
CPT = 81
E_PAD = CPT * NW * CHUNK
RING = 3
T_OUTER = CPT // RING
ROW_BLK = 80
NROW_BLKS = N // ROW_BLK


def _spmm_sc(y, sw_r, dst_r):
    mesh = plsc.VectorSubcoreMesh(core_axis_name="c", subcore_axis_name="s")
    cp = pltpu.CompilerParams()
    if "needs_layout_passes" in pltpu.CompilerParams.__dataclass_fields__:
        cp = dataclasses.replace(cp, needs_layout_passes=False)

    @functools.partial(
        pl.kernel,
        mesh=mesh,
        compiler_params=cp,
        out_type=jax.ShapeDtypeStruct((NC, N, D), jnp.float32),
        scratch_types=(
            [pltpu.VMEM_SHARED((N, D), jnp.float32)]
            + [pltpu.VMEM((2, CHUNK), jnp.int32)] * RING
            + [pltpu.VMEM((CHUNK,), jnp.int32)] * RING
            + [pltpu.VMEM((CHUNK, D), jnp.float32)] * RING
            + [pltpu.SemaphoreType.DMA] * (4 * RING)
        ),
    )
    def spmm_kernel(y_hbm, sw_hbm, dst_hbm, out_hbm, acc_sh, *scr):
        swbufs = scr[0:RING]
        dbufs = scr[RING:2 * RING]
        rows = scr[2 * RING:3 * RING]
        gsems = scr[3 * RING:4 * RING]
        ssems = scr[4 * RING:5 * RING]
        isems = scr[5 * RING:6 * RING]
        dsems = scr[6 * RING:7 * RING]

        c = lax.axis_index("c")
        s = lax.axis_index("s")
        wid = s * NC + c

        def issue_sw(k, j):
            pltpu.async_copy(sw_hbm.at[wid, k], swbufs[j], isems[j])

        def wait_sw(j):
            pltpu.make_async_copy(sw_hbm.at[0, 0], swbufs[j], isems[j]).wait()

        def issue_dst(k, j):
            pltpu.async_copy(dst_hbm.at[wid, k], dbufs[j], dsems[j])

        def wait_dst(j):
            pltpu.make_async_copy(dst_hbm.at[0, 0], dbufs[j], dsems[j]).wait()

        def issue_gather(j):
            pltpu.async_copy(y_hbm.at[swbufs[j].at[0]], rows[j], gsems[j])

        def wait_gather(j):
            pltpu.make_async_copy(y_hbm.at[swbufs[j].at[0]], rows[j],
                                  gsems[j]).wait()

        def issue_scatter(j):
            pltpu.async_copy(rows[j], acc_sh.at[dbufs[j]], ssems[j], add=True)

        def wait_scatter(j):
            pltpu.make_async_copy(rows[j], acc_sh.at[dbufs[j]], ssems[j]).wait()

        def scale(j):
            rj, swj = rows[j], swbufs[j]

            @pl.loop(0, CHUNK // 16)
            def _(g):
                wv16 = plsc.bitcast(swj.at[1, pl.ds(g * 16, 16)][...],
                                    jnp.float32)
                e0 = g * 16
                for i in range(16):
                    wv = jnp.zeros((16,), jnp.float32) + wv16[i]
                    for fb in range(D // 16):
                        sl = pl.ds(fb * 16, 16)
                        rj.at[e0 + i, sl][...] = rj.at[e0 + i, sl][...] * wv

        @pl.loop(0, ROW_BLK)
        def _(r):
            for kk in range(D // 16):
                rows[0].at[r, pl.ds(kk * 16, 16)][...] = (
                    jnp.zeros((16,), jnp.float32))

        @pl.loop(s, NROW_BLKS, step=NS)
        def _(blk):
            pltpu.sync_copy(rows[0].at[pl.ds(0, ROW_BLK)],
                            acc_sh.at[pl.ds(blk * ROW_BLK, ROW_BLK)])
        plsc.subcore_barrier()

        for m in range(2):
            issue_sw(m, m)
        issue_dst(0, 0)
        wait_sw(0)
        issue_gather(0)

        @pl.loop(0, T_OUTER)
        def _(t):
            for j in range(RING):
                jn = (j + 1) % RING
                k = RING * t + j

                def look_ahead():
                    def drain():
                        wait_scatter(jn)
                    if j >= 2:
                        drain()
                    else:
                        pl.when(t > 0)(drain)
                    issue_dst(k + 1, jn)
                    wait_sw(jn)
                    issue_gather(jn)

                if j >= 2:
                    pl.when(t < T_OUTER - 1)(look_ahead)
                else:
                    look_ahead()

                def stage_sw():
                    issue_sw(k + 2, (j + 2) % RING)
                if j >= 1:
                    pl.when(t < T_OUTER - 1)(stage_sw)
                else:
                    stage_sw()

                wait_gather(j)
                scale(j)
                wait_dst(j)
                issue_scatter(j)

        for j in range(RING):
            wait_scatter(j)
        plsc.subcore_barrier()

        @pl.loop(s, NROW_BLKS, step=NS)
        def _(blk):
            rr = blk * ROW_BLK
            pltpu.sync_copy(acc_sh.at[pl.ds(rr, ROW_BLK)],
                            out_hbm.at[c, pl.ds(rr, ROW_BLK)])

    return spmm_kernel(y, sw_r, dst_r)


def _mm_first(x, W1):
    def body(x_ref, w_ref, o_ref):
        o_ref[...] = jnp.dot(x_ref[...], w_ref[...],
                             preferred_element_type=jnp.float32)
    return pl.pallas_call(
        body, out_shape=jax.ShapeDtypeStruct((N, D), jnp.float32))(x, W1)


def _mm_mid(p, b, W):
    def body(p_ref, b_ref, w_ref, o_ref):
        h = jnp.maximum(p_ref[0] + p_ref[1] + b_ref[...], 0.0)
        o_ref[...] = jnp.dot(h, w_ref[...], preferred_element_type=jnp.float32)
    return pl.pallas_call(
        body, out_shape=jax.ShapeDtypeStruct((N, D), jnp.float32))(p, b, W)


def _mm_last(p, b, W_out, b_out):
    def body(p_ref, b_ref, w_ref, bo_ref, o_ref):
        h = jnp.maximum(p_ref[0] + p_ref[1] + b_ref[...], 0.0)
        z = jnp.dot(h, w_ref[...], preferred_element_type=jnp.float32)
        o_ref[...] = z[:, 0] + bo_ref[0]
    return pl.pallas_call(
        body, out_shape=jax.ShapeDtypeStruct((N,), jnp.float32))(
            p, b, W_out, b_out)


def kernel(x, edge_index, edge_weight, W1, b1, W2, b2, W_out, b_out):
    pad = E_PAD - E
    ei = edge_index.astype(jnp.int32)
    src_r = jnp.concatenate(
        [ei[0], jnp.zeros((pad,), jnp.int32)]).reshape(NW, CPT, CHUNK)
    dst_r = jnp.concatenate(
        [ei[1], jnp.zeros((pad,), jnp.int32)]).reshape(NW, CPT, CHUNK)
    w_bits = jax.lax.bitcast_convert_type(
        jnp.concatenate(
            [edge_weight.astype(jnp.float32), jnp.zeros((pad,), jnp.float32)]),
        jnp.int32).reshape(NW, CPT, CHUNK)
    sw_r = jnp.stack([src_r, w_bits], axis=2)

    y1 = _mm_first(x, W1)
    p1 = _spmm_sc(y1, sw_r, dst_r)
    y2 = _mm_mid(p1, b1, W2)
    p2 = _spmm_sc(y2, sw_r, dst_r)
    return _mm_last(p2, b2, W_out, b_out)


# speedup vs baseline: 2.5767x; 2.5767x over previous
import dataclasses
import functools

import jax
import jax.numpy as jnp
from jax import lax
from jax.experimental import pallas as pl
from jax.experimental.pallas import tpu as pltpu
from jax.experimental.pallas import tpu_sc as plsc

N = 10000
E = 320000
D = 128
CHUNK = 112
NC, NS = 2, 16
NW = NC * NS
CPT = 90
E_PAD = CPT * NW * CHUNK
RING = 3
T_OUTER = CPT // RING
ROW_BLK = 80
NROW_BLKS = N // ROW_BLK


def _spmm_sc(y, sw_r, dst_r):
    mesh = plsc.VectorSubcoreMesh(core_axis_name="c", subcore_axis_name="s")
    cp = pltpu.CompilerParams()
    if "needs_layout_passes" in pltpu.CompilerParams.__dataclass_fields__:
        cp = dataclasses.replace(cp, needs_layout_passes=False)

    @functools.partial(
        pl.kernel,
        mesh=mesh,
        compiler_params=cp,
        out_type=jax.ShapeDtypeStruct((NC, N, D), jnp.float32),
        scratch_types=(
            [pltpu.VMEM_SHARED((N, D), jnp.float32)]
            + [pltpu.VMEM((2, CHUNK), jnp.int32)] * RING
            + [pltpu.VMEM((CHUNK,), jnp.int32)] * RING
            + [pltpu.VMEM((CHUNK, D), jnp.float32)] * RING
            + [pltpu.SemaphoreType.DMA] * (4 * RING)
        ),
    )
    def spmm_kernel(y_hbm, sw_hbm, dst_hbm, out_hbm, acc_sh, *scr):
        swbufs = scr[0:RING]
        dbufs = scr[RING:2 * RING]
        rows = scr[2 * RING:3 * RING]
        gsems = scr[3 * RING:4 * RING]
        ssems = scr[4 * RING:5 * RING]
        isems = scr[5 * RING:6 * RING]
        dsems = scr[6 * RING:7 * RING]

        c = lax.axis_index("c")
        s = lax.axis_index("s")
        wid = s * NC + c

        def issue_sw(k, j):
            pltpu.async_copy(sw_hbm.at[wid, k], swbufs[j], isems[j])

        def wait_sw(j):
            pltpu.make_async_copy(sw_hbm.at[0, 0], swbufs[j], isems[j]).wait()

        def issue_dst(k, j):
            pltpu.async_copy(dst_hbm.at[wid, k], dbufs[j], dsems[j])

        def wait_dst(j):
            pltpu.make_async_copy(dst_hbm.at[0, 0], dbufs[j], dsems[j]).wait()

        def issue_gather(j):
            pltpu.async_copy(y_hbm.at[swbufs[j].at[0]], rows[j], gsems[j])

        def wait_gather(j):
            pltpu.make_async_copy(y_hbm.at[swbufs[j].at[0]], rows[j],
                                  gsems[j]).wait()

        def issue_scatter(j):
            pltpu.async_copy(rows[j], acc_sh.at[dbufs[j]], ssems[j], add=True)

        def wait_scatter(j):
            pltpu.make_async_copy(rows[j], acc_sh.at[dbufs[j]], ssems[j]).wait()

        def scale(j):
            rj, swj = rows[j], swbufs[j]

            @pl.loop(0, CHUNK // 16)
            def _(g):
                wv16 = plsc.bitcast(swj.at[1, pl.ds(g * 16, 16)][...],
                                    jnp.float32)
                e0 = g * 16
                for i in range(16):
                    wv = jnp.zeros((16,), jnp.float32) + wv16[i]
                    for fb in range(D // 16):
                        sl = pl.ds(fb * 16, 16)
                        rj.at[e0 + i, sl][...] = rj.at[e0 + i, sl][...] * wv

        @pl.loop(0, ROW_BLK)
        def _(r):
            for kk in range(D // 16):
                rows[0].at[r, pl.ds(kk * 16, 16)][...] = (
                    jnp.zeros((16,), jnp.float32))

        @pl.loop(s, NROW_BLKS, step=NS)
        def _(blk):
            pltpu.sync_copy(rows[0].at[pl.ds(0, ROW_BLK)],
                            acc_sh.at[pl.ds(blk * ROW_BLK, ROW_BLK)])
        plsc.subcore_barrier()

        for m in range(2):
            issue_sw(m, m)
        issue_dst(0, 0)
        wait_sw(0)
        issue_gather(0)

        @pl.loop(0, T_OUTER)
        def _(t):
            for j in range(RING):
                jn = (j + 1) % RING
                k = RING * t + j

                def look_ahead():
                    def drain():
                        wait_scatter(jn)
                    if j >= 2:
                        drain()
                    else:
                        pl.when(t > 0)(drain)
                    issue_dst(k + 1, jn)
                    wait_sw(jn)
                    issue_gather(jn)

                if j >= 2:
                    pl.when(t < T_OUTER - 1)(look_ahead)
                else:
                    look_ahead()

                def stage_sw():
                    issue_sw(k + 2, (j + 2) % RING)
                if j >= 1:
                    pl.when(t < T_OUTER - 1)(stage_sw)
                else:
                    stage_sw()

                wait_gather(j)
                scale(j)
                wait_dst(j)
                issue_scatter(j)

        for j in range(RING):
            wait_scatter(j)
        plsc.subcore_barrier()

        @pl.loop(s, NROW_BLKS, step=NS)
        def _(blk):
            rr = blk * ROW_BLK
            pltpu.sync_copy(acc_sh.at[pl.ds(rr, ROW_BLK)],
                            out_hbm.at[c, pl.ds(rr, ROW_BLK)])

    return spmm_kernel(y, sw_r, dst_r)


def _mm_first(x, W1):
    def body(x_ref, w_ref, o_ref):
        o_ref[...] = jnp.dot(x_ref[...], w_ref[...],
                             preferred_element_type=jnp.float32)
    return pl.pallas_call(
        body, out_shape=jax.ShapeDtypeStruct((N, D), jnp.float32))(x, W1)


def _mm_mid(p, b, W):
    def body(p_ref, b_ref, w_ref, o_ref):
        h = jnp.maximum(p_ref[0] + p_ref[1] + b_ref[...], 0.0)
        o_ref[...] = jnp.dot(h, w_ref[...], preferred_element_type=jnp.float32)
    return pl.pallas_call(
        body, out_shape=jax.ShapeDtypeStruct((N, D), jnp.float32))(p, b, W)


def _mm_last(p, b, W_out, b_out):
    def body(p_ref, b_ref, w_ref, bo_ref, o_ref):
        h = jnp.maximum(p_ref[0] + p_ref[1] + b_ref[...], 0.0)
        z = jnp.dot(h, w_ref[...], preferred_element_type=jnp.float32)
        o_ref[...] = z[:, 0] + bo_ref[0]
    return pl.pallas_call(
        body, out_shape=jax.ShapeDtypeStruct((N,), jnp.float32))(
            p, b, W_out, b_out)


def kernel(x, edge_index, edge_weight, W1, b1, W2, b2, W_out, b_out):
    pad = E_PAD - E
    ei = edge_index.astype(jnp.int32)
    src_r = jnp.concatenate(
        [ei[0], jnp.zeros((pad,), jnp.int32)]).reshape(NW, CPT, CHUNK)
    dst_r = jnp.concatenate(
        [ei[1], jnp.zeros((pad,), jnp.int32)]).reshape(NW, CPT, CHUNK)
    w_bits = jax.lax.bitcast_convert_type(
        jnp.concatenate(
            [edge_weight.astype(jnp.float32), jnp.zeros((pad,), jnp.float32)]),
        jnp.int32).reshape(NW, CPT, CHUNK)
    sw_r = jnp.stack([src_r, w_bits], axis=2)

    y1 = _mm_first(x, W1)
    p1 = _spmm_sc(y1, sw_r, dst_r)
    y2 = _mm_mid(p1, b1, W2)
    p2 = _spmm_sc(y2, sw_r, dst_r)
    return _mm_last(p2, b2, W_out, b_out)
